# baseline (speedup 1.0000x reference)
import jax
import jax.numpy as jnp
import numpy as np
from jax import lax
from jax.experimental import pallas as pl
from jax.experimental.pallas import tpu as pltpu, tpu_sc as plsc

CHAR = 256
B = 4096
S = 50
NW = 32
CPW = B // NW
NG = CPW // 16

_BL_TAB = np.arange(CPW, dtype=np.int32)
_WZ_POS = np.asarray([1, 1] + list(range(2, 16)), np.int32)


def _sc_body(xt_hbm, blt_hbm, wz_hbm, out_hbm,
             idx0, idx1, bl_v, wz_v, buf0, buf1, sem0, sem1):
    c = lax.axis_index("c")
    s = lax.axis_index("s")
    wid = s * 2 + c
    col0 = wid * CPW

    zeros16 = jnp.zeros((16,), jnp.int32)
    ones16 = jnp.ones((16,), jnp.int32)

    pltpu.sync_copy(blt_hbm, bl_v)
    pltpu.sync_copy(wz_hbm, wz_v)

    @pl.loop(0, CPW)
    def _zrow(r):
        for cg in range(CHAR // 16):
            buf0[r, pl.ds(cg * 16, 16)] = zeros16
            buf1[r, pl.ds(cg * 16, 16)] = zeros16

    def _scatter_ones(buf, idx_v):
        @pl.when(wid == 0)
        def _g0_skip_col0():
            wz = wz_v[pl.ds(0, 16)]
            xv = plsc.load_gather(idx_v, [wz])
            plsc.store_scatter(buf, [wz, xv], ones16)

        @pl.when(wid != 0)
        def _g0_normal():
            plsc.store_scatter(
                buf, [bl_v[pl.ds(0, 16)], idx_v[pl.ds(0, 16)]], ones16)

        for g in range(1, NG):
            sl16 = pl.ds(g * 16, 16)
            plsc.store_scatter(buf, [bl_v[sl16], idx_v[sl16]], ones16)

    def _scatter_zeros(buf, idx_v):
        for g in range(NG):
            sl16 = pl.ds(g * 16, 16)
            plsc.store_scatter(buf, [bl_v[sl16], idx_v[sl16]], zeros16)

    def _slot(k, par, buf, idx_v, sem):
        srow = 2 * k + par
        dst = out_hbm.at[srow, pl.ds(col0, CPW)]

        @pl.when(k > 0)
        def _recycle():
            pltpu.make_async_copy(buf, dst, sem).wait()
            _scatter_zeros(buf, idx_v)

        pltpu.sync_copy(xt_hbm.at[pl.ds(srow * B + col0, CPW)], idx_v)
        _scatter_ones(buf, idx_v)
        pltpu.async_copy(buf, dst, sem)

    @pl.loop(0, S // 2)
    def _chunk(k):
        _slot(k, 0, buf0, idx0, sem0)
        _slot(k, 1, buf1, idx1, sem1)

    pltpu.make_async_copy(buf0, out_hbm.at[0, pl.ds(col0, CPW)], sem0).wait()
    pltpu.make_async_copy(buf1, out_hbm.at[0, pl.ds(col0, CPW)], sem1).wait()


def kernel(x):
    xt = x.T.reshape((S * B,))
    mesh = plsc.VectorSubcoreMesh(core_axis_name="c", subcore_axis_name="s")
    out_t = pl.kernel(
        _sc_body,
        mesh=mesh,
        compiler_params=pltpu.CompilerParams(needs_layout_passes=False),
        out_type=jax.ShapeDtypeStruct((S, B, CHAR), jnp.int32),
        scratch_types=[
            pltpu.VMEM((CPW,), jnp.int32),
            pltpu.VMEM((CPW,), jnp.int32),
            pltpu.VMEM((CPW,), jnp.int32),
            pltpu.VMEM((16,), jnp.int32),
            pltpu.VMEM((CPW, CHAR), jnp.int32),
            pltpu.VMEM((CPW, CHAR), jnp.int32),
            pltpu.SemaphoreType.DMA,
            pltpu.SemaphoreType.DMA,
        ],
    )(xt, jnp.asarray(_BL_TAB), jnp.asarray(_WZ_POS))
    return jnp.swapaxes(out_t, 0, 1)


# speedup vs baseline: 1.0402x; 1.0402x over previous
import jax
import jax.numpy as jnp
import numpy as np
from jax import lax
from jax.experimental import pallas as pl
from jax.experimental.pallas import tpu as pltpu, tpu_sc as plsc

CHAR = 256
B = 4096
S = 50
NW = 32
CPW = B // NW
NG = CPW // 16

_BL_TAB = np.arange(CPW, dtype=np.int32)
_WZ_POS = np.asarray([1, 1] + list(range(2, 16)), np.int32)


def _sc_body(xt_hbm, blt_hbm, wz_hbm, out_hbm,
             codes, bl_v, wz_v, buf0, buf1, sem0, sem1):
    c = lax.axis_index("c")
    s = lax.axis_index("s")
    wid = s * 2 + c
    col0 = wid * CPW

    zeros16 = jnp.zeros((16,), jnp.int32)
    ones16 = jnp.ones((16,), jnp.int32)

    pltpu.sync_copy(blt_hbm, bl_v)
    pltpu.sync_copy(wz_hbm, wz_v)
    pltpu.sync_copy(xt_hbm.at[:, pl.ds(col0, CPW)], codes)

    @pl.loop(0, CPW)
    def _zrow(r):
        for cg in range(CHAR // 16):
            buf0[r, pl.ds(cg * 16, 16)] = zeros16
            buf1[r, pl.ds(cg * 16, 16)] = zeros16

    def _scatter(buf, srow, val16, skip_col0):
        for g in range(NG):
            sl16 = pl.ds(g * 16, 16)
            if g == 0 and skip_col0:
                @pl.when(wid == 0)
                def _g0_dup():
                    wz = wz_v[pl.ds(0, 16)]
                    xv = plsc.load_gather(codes, [jnp.full((16,), srow, jnp.int32), wz])
                    plsc.store_scatter(buf, [wz, xv], val16)

                @pl.when(wid != 0)
                def _g0_normal():
                    plsc.store_scatter(
                        buf, [bl_v[sl16], codes[srow, sl16]], val16)
            else:
                plsc.store_scatter(buf, [bl_v[sl16], codes[srow, sl16]], val16)

    def _slot(k, par, buf, sem):
        srow = 2 * k + par
        dst = out_hbm.at[srow, pl.ds(col0, CPW)]

        @pl.when(k > 0)
        def _recycle():
            pltpu.make_async_copy(buf, dst, sem).wait()
            _scatter(buf, srow - 2, zeros16, skip_col0=False)

        _scatter(buf, srow, ones16, skip_col0=True)
        pltpu.async_copy(buf, dst, sem)

    @pl.loop(0, S // 2)
    def _chunk(k):
        _slot(k, 0, buf0, sem0)
        _slot(k, 1, buf1, sem1)

    pltpu.make_async_copy(buf0, out_hbm.at[0, pl.ds(col0, CPW)], sem0).wait()
    pltpu.make_async_copy(buf1, out_hbm.at[0, pl.ds(col0, CPW)], sem1).wait()


def kernel(x):
    xt = x.T
    mesh = plsc.VectorSubcoreMesh(core_axis_name="c", subcore_axis_name="s")
    out_t = pl.kernel(
        _sc_body,
        mesh=mesh,
        compiler_params=pltpu.CompilerParams(needs_layout_passes=False),
        out_type=jax.ShapeDtypeStruct((S, B, CHAR), jnp.int32),
        scratch_types=[
            pltpu.VMEM((S, CPW), jnp.int32),
            pltpu.VMEM((CPW,), jnp.int32),
            pltpu.VMEM((16,), jnp.int32),
            pltpu.VMEM((CPW, CHAR), jnp.int32),
            pltpu.VMEM((CPW, CHAR), jnp.int32),
            pltpu.SemaphoreType.DMA,
            pltpu.SemaphoreType.DMA,
        ],
    )(xt, jnp.asarray(_BL_TAB), jnp.asarray(_WZ_POS))
    return jnp.swapaxes(out_t, 0, 1)
